# baseline (speedup 1.0000x reference)
import functools

import jax
import jax.numpy as jnp
from jax import lax
from jax.experimental import pallas as pl
from jax.experimental.pallas import tpu as pltpu
from jax.experimental.pallas import tpu_sc as plsc

V = 100000
D = 128
B = 327680
NC = 2
NS = 16
NW = NC * NS
BPW = B // NW
CH = 128
NCHUNK = BPW // CH
NBUF = 5
L = 3
NG = NCHUNK // NBUF

_mesh = plsc.VectorSubcoreMesh(core_axis_name="c", subcore_axis_name="s")


@functools.partial(
    pl.kernel,
    out_type=jax.ShapeDtypeStruct((B, D), jnp.float32),
    mesh=_mesh,
    scratch_types=[
        pltpu.VMEM((NCHUNK, CH), jnp.int32),
    ]
    + [pltpu.VMEM((CH, D), jnp.float32) for _ in range(NBUF)]
    + [pltpu.SemaphoreType.DMA for _ in range(2 * NBUF)],
)
def _gather_kernel(table_hbm, idx_hbm, out_hbm, idx_v, *rest):
    bufs = list(rest[:NBUF])
    sin = list(rest[NBUF:2 * NBUF])
    sout = list(rest[2 * NBUF:])

    wid = lax.axis_index("c") * NS + lax.axis_index("s")
    base = wid * BPW
    pltpu.sync_copy(idx_hbm.at[wid], idx_v)

    def start_gather(g, slot):
        pltpu.async_copy(table_hbm.at[idx_v.at[g]], bufs[slot], sin[slot])

    def wait_gather(slot):
        pltpu.make_async_copy(
            table_hbm.at[idx_v.at[0]], bufs[slot], sin[slot]).wait()

    def start_out(g, slot):
        pltpu.async_copy(
            bufs[slot], out_hbm.at[pl.ds(base + g * CH, CH)], sout[slot])

    def wait_out(slot):
        pltpu.make_async_copy(
            bufs[slot], out_hbm.at[pl.ds(base, CH)], sout[slot]).wait()

    for g in range(L):
        start_gather(g, g % NBUF)
    for g in range(NBUF):
        wait_gather(g)
        start_out(g, g)
        s = (g + L) % NBUF
        if g + L >= NBUF:
            wait_out(s)
        start_gather(g + L, s)

    def outer(i0, carry):
        for b in range(NBUF):
            g = i0 * NBUF + b
            s = (b + L) % NBUF
            wait_gather(b)
            start_out(g, b)
            wait_out(s)
            start_gather(g + L, s)
        return carry

    lax.fori_loop(1, NG - 1, outer, 0)

    t = (NG - 1) * NBUF
    for b in range(NBUF):
        g = t + b
        wait_gather(b)
        start_out(g, b)
        if g + L < NCHUNK:
            s = (b + L) % NBUF
            wait_out(s)
            start_gather(g + L, s)
    for b in range(NBUF):
        wait_out(b)


def kernel(lower_voxel, unq_inv):
    idx = unq_inv.reshape(NW, NCHUNK, CH).astype(jnp.int32)
    return _gather_kernel(lower_voxel, idx)
